# baseline (speedup 1.0000x reference)
import functools

import jax
import jax.numpy as jnp
from jax import lax
from jax.experimental import pallas as pl
from jax.experimental.pallas import tpu as pltpu
from jax.experimental.pallas import tpu_sc as plsc

_ITRS = 7
_NC = 2
_NT = 16
_K = 128
_RSUB = 32


def _mesh():
  return plsc.VectorSubcoreMesh(
      core_axis_name="c", subcore_axis_name="s", num_cores=_NC)


def _make_phase0(N, D, E):
  NCH = E // _K
  HN = N // _NC
  RPT = HN // _NT
  NSUB = RPT // _RSUB
  CSL = D // 16

  @functools.partial(
      pl.kernel,
      out_type=(
          jax.ShapeDtypeStruct((N, D), jnp.float32),
          jax.ShapeDtypeStruct((N, D), jnp.float32),
          jax.ShapeDtypeStruct((N, D), jnp.float32),
      ),
      mesh=_mesh(),
      scratch_types=[
          pltpu.VMEM((4, _K, D), jnp.float32),
          pltpu.VMEM((4, 1, _K), jnp.int32),
          pltpu.VMEM((_RSUB, D), jnp.float32),
          pltpu.VMEM((_RSUB, D), jnp.float32),
          pltpu.VMEM((_RSUB, D), jnp.float32),
          pltpu.VMEM_SHARED((HN + 8, D), jnp.float32),
          pltpu.SemaphoreType.DMA,
          pltpu.SemaphoreType.DMA,
          pltpu.SemaphoreType.DMA,
          pltpu.SemaphoreType.DMA,
          pltpu.SemaphoreType.DMA,
          pltpu.SemaphoreType.DMA,
          pltpu.SemaphoreType.DMA,
          pltpu.SemaphoreType.DMA,
          pltpu.SemaphoreType.DMA,
          pltpu.SemaphoreType.DMA,
          pltpu.SemaphoreType.DMA,
          pltpu.SemaphoreType.DMA,
      ],
  )
  def k0(p_hbm, rel_hbm, dst3d_hbm, q_hbm, ht_hbm, cum_hbm,
         rowb, dstrow, bufA, bufQ, zbuf, agg,
         semA0, semA1, semA2, semA3, semB0, semB1, semB2, semB3,
         semC0, semC1, semC2, semC3):
    cid = lax.axis_index("c")
    w = lax.axis_index("s")
    semA = (semA0, semA1, semA2, semA3)
    semB = (semB0, semB1, semB2, semB3)
    semC = (semC0, semC1, semC2, semC3)

    def add_into(dst_ref, src_ref):
      def row(r, carry):
        for l in range(CSL):
          sl = pl.ds(l * 16, 16)
          dst_ref[r, sl] = dst_ref[r, sl] + src_ref[r, sl]
        return carry
      lax.fori_loop(0, _RSUB, row, 0)

    def zrow(r, carry):
      for l in range(CSL):
        zbuf[r, pl.ds(l * 16, 16)] = jnp.zeros((16,), jnp.float32)
      return carry
    lax.fori_loop(0, _RSUB, zrow, 0)

    def zchunk(s_, carry):
      pltpu.sync_copy(zbuf, agg.at[pl.ds(w * RPT + s_ * _RSUB, _RSUB)])
      return carry
    lax.fori_loop(0, NSUB, zchunk, 0)

    plsc.subcore_barrier()

    n_my = (NCH - w + _NT - 1) // _NT

    def rel_issue(i, u):
      j = w + i * _NT
      pltpu.async_copy(rel_hbm.at[pl.ds(j * _K, _K)], rowb.at[u], semA[u])
      pltpu.async_copy(dst3d_hbm.at[cid, j], dstrow.at[u], semB[u])

    def rel_wait(i, u):
      j = w + i * _NT
      pltpu.make_async_copy(
          rel_hbm.at[pl.ds(j * _K, _K)], rowb.at[u], semA[u]).wait()
      pltpu.make_async_copy(
          dst3d_hbm.at[cid, j], dstrow.at[u], semB[u]).wait()

    def rel_scat_issue(u):
      pltpu.async_copy(
          rowb.at[u],
          agg.at[plsc.Indices(dstrow.at[u, 0], ignored_value=-1)],
          semC[u], add=True)

    def rel_scat_wait(u):
      pltpu.make_async_copy(
          rowb.at[u],
          agg.at[plsc.Indices(dstrow.at[u, 0], ignored_value=-1)],
          semC[u]).wait()

    for u in range(4):
      @pl.when(u < n_my)
      def _():
        rel_issue(u, u)

    def relgroup(g, carry):
      for u in range(4):
        c = 4 * g + u
        @pl.when(c < n_my)
        def _():
          rel_wait(c, u)
          rel_scat_issue(u)
      for u in range(4):
        c4 = 4 * g + u + 4
        @pl.when(c4 < n_my)
        def _():
          rel_scat_wait(u)
          rel_issue(c4, u)
      return carry
    lax.fori_loop(0, (n_my + 3) // 4, relgroup, 0)

    for u in range(4):
      @pl.when(u < n_my)
      def _():
        rel_scat_wait(u)

    plsc.subcore_barrier()

    def initchunk(s_, carry):
      rows = pl.ds(w * RPT + s_ * _RSUB, _RSUB)
      grows = pl.ds(cid * HN + w * RPT + s_ * _RSUB, _RSUB)
      pltpu.sync_copy(p_hbm.at[grows], bufQ)
      pltpu.sync_copy(agg.at[rows], bufA)
      pltpu.sync_copy(bufQ, ht_hbm.at[grows])
      add_into(bufQ, bufA)
      pltpu.sync_copy(bufQ, q_hbm.at[grows])
      pltpu.sync_copy(zbuf, cum_hbm.at[grows])
      return carry
    lax.fori_loop(0, NSUB, initchunk, 0)

  return k0


def _make_iter(N, D, C):
  HN = N // _NC
  RPT = HN // _NT
  NSUB = RPT // _RSUB
  CSL = D // 16
  RS = 16
  NSUB2 = RPT // RS
  assert C % 5 == 0

  @functools.partial(
      pl.kernel,
      out_type=(
          jax.ShapeDtypeStruct((N, D), jnp.float32),
          jax.ShapeDtypeStruct((N, D), jnp.float32),
      ),
      mesh=_mesh(),
      scratch_types=[
          pltpu.VMEM((5, 2, _K), jnp.int32),
          pltpu.VMEM((5, _K, D), jnp.float32),
          pltpu.VMEM((RS, D), jnp.float32),
          pltpu.VMEM((RS, D), jnp.float32),
          pltpu.VMEM((RS, D), jnp.float32),
          pltpu.VMEM_SHARED((HN + 8, D), jnp.float32),
          pltpu.SemaphoreType.DMA,
          pltpu.SemaphoreType.DMA,
          pltpu.SemaphoreType.DMA,
          pltpu.SemaphoreType.DMA,
          pltpu.SemaphoreType.DMA,
          pltpu.SemaphoreType.DMA,
          pltpu.SemaphoreType.DMA,
          pltpu.SemaphoreType.DMA,
          pltpu.SemaphoreType.DMA,
          pltpu.SemaphoreType.DMA,
          pltpu.SemaphoreType.DMA,
          pltpu.SemaphoreType.DMA,
          pltpu.SemaphoreType.DMA,
          pltpu.SemaphoreType.DMA,
          pltpu.SemaphoreType.DMA,
          pltpu.SemaphoreType.DMA,
      ],
  )
  def kit(edg_hbm, q_hbm, ht_hbm, cum_hbm,
          htn_hbm, cumn_hbm,
          idxb, rowb, bufA, bufQ, zbuf, agg,
          semG0, semG1, semG2, semG3, semG4, semI0, semI1, semI2, semI3,
          semI4, semS0, semS1, semS2, semS3, semS4, semW):
    cid = lax.axis_index("c")
    w = lax.axis_index("s")
    semG = (semG0, semG1, semG2, semG3, semG4)
    semI = (semI0, semI1, semI2, semI3, semI4)
    semS = (semS0, semS1, semS2, semS3, semS4)

    def add_into(dst_ref, src_ref):
      def row(r, carry):
        for l in range(CSL):
          sl = pl.ds(l * 16, 16)
          dst_ref[r, sl] = dst_ref[r, sl] + src_ref[r, sl]
        return carry
      lax.fori_loop(0, RS, row, 0)

    def zrow(r, carry):
      for l in range(CSL):
        zbuf[r, pl.ds(l * 16, 16)] = jnp.zeros((16,), jnp.float32)
      return carry
    lax.fori_loop(0, RS, zrow, 0)

    def zchunk(s_, carry):
      pltpu.sync_copy(zbuf, agg.at[pl.ds(w * RPT + s_ * RS, RS)])
      return carry
    lax.fori_loop(0, NSUB2, zchunk, 0)

    plsc.subcore_barrier()

    def idx_issue(c, u):
      pltpu.async_copy(edg_hbm.at[cid, w, c], idxb.at[u], semI[u])

    def idx_wait(c, u):
      pltpu.make_async_copy(edg_hbm.at[cid, w, c], idxb.at[u], semI[u]).wait()

    def g_issue(u):
      pltpu.async_copy(
          ht_hbm.at[plsc.Indices(idxb.at[u, 0], ignored_value=-1)],
          rowb.at[u], semG[u])

    def g_wait(u):
      pltpu.make_async_copy(
          ht_hbm.at[plsc.Indices(idxb.at[u, 0], ignored_value=-1)],
          rowb.at[u], semG[u]).wait()

    def scat_issue(u):
      pltpu.async_copy(
          rowb.at[u],
          agg.at[plsc.Indices(idxb.at[u, 1], ignored_value=-1)],
          semS[u], add=True)

    def scat_wait(u):
      pltpu.make_async_copy(
          rowb.at[u],
          agg.at[plsc.Indices(idxb.at[u, 1], ignored_value=-1)],
          semS[u]).wait()

    for u in range(5):
      idx_issue(u, u)
    for u in range(5):
      idx_wait(u, u)
      g_issue(u)

    def group(g, carry):
      for u in range(5):
        g_wait(u)
        scat_issue(u)
      for u in range(5):
        c5 = 5 * g + u + 5
        @pl.when(c5 < C)
        def _():
          scat_wait(u)
          idx_issue(c5, u)
          idx_wait(c5, u)
          g_issue(u)
      return carry
    lax.fori_loop(0, C // 5, group, 0)

    for u in range(5):
      scat_wait(u)

    plsc.subcore_barrier()

    def wait_writes(sp):
      prows = pl.ds(cid * HN + w * RPT + sp * RS, RS)
      pltpu.make_async_copy(bufQ, htn_hbm.at[prows], semW).wait()
      pltpu.make_async_copy(bufA, cumn_hbm.at[prows], semW).wait()

    def drain(s_, carry2):
      rows = pl.ds(w * RPT + s_ * RS, RS)
      grows = pl.ds(cid * HN + w * RPT + s_ * RS, RS)
      @pl.when(s_ > 0)
      def _():
        wait_writes(s_ - 1)
      pltpu.sync_copy(agg.at[rows], bufA)
      pltpu.sync_copy(q_hbm.at[grows], bufQ)
      add_into(bufQ, bufA)
      pltpu.async_copy(bufQ, htn_hbm.at[grows], semW)
      pltpu.sync_copy(cum_hbm.at[grows], bufA)
      add_into(bufA, bufQ)
      pltpu.async_copy(bufA, cumn_hbm.at[grows], semW)
      return carry2
    lax.fori_loop(0, NSUB2, drain, 0)
    wait_writes(NSUB2 - 1)

  return kit


def kernel(particle_effect, relation_effect, edges):
  N, D = particle_effect.shape
  E = relation_effect.shape[0]
  assert E % _K == 0 and D % 16 == 0

  src = edges[0].astype(jnp.int32)
  dst = edges[1].astype(jnp.int32)

  NP = -(-N // (_NC * _NT * _RSUB)) * (_NC * _NT * _RSUB)
  HN = NP // _NC
  p_pad = jnp.pad(particle_effect, ((0, NP - N), (0, 0))) if NP != N \
      else particle_effect

  C = -(-E // (_NT * _K))
  C = -(-C // 8) * 8
  tot = _NT * C * _K
  pad = tot - E
  if pad:
    neg = -jnp.ones((pad,), jnp.int32)
    src_p = jnp.concatenate([src, neg])
    dst_p = jnp.concatenate([dst, neg])
  else:
    src_p, dst_p = src, dst

  own0 = (dst_p >= 0) & (dst_p < HN)
  own1 = dst_p >= HN
  def tbl(ownm, vs, vd):
    return jnp.stack([
        jnp.where(ownm, vs, -1).reshape(_NT, C, _K),
        jnp.where(ownm, vd, -1).reshape(_NT, C, _K),
    ], axis=2)
  edg = jnp.stack([tbl(own0, src_p, dst_p), tbl(own1, src_p, dst_p - HN)])

  o0 = dst < HN
  dst3d = jnp.stack([
      jnp.where(o0, dst, -1).reshape(E // _K, 1, _K),
      jnp.where(o0, -1, dst - HN).reshape(E // _K, 1, _K),
  ])

  q, ht, cum = _make_phase0(NP, D, E)(p_pad, relation_effect, dst3d)
  step = _make_iter(NP, D, C)
  for _ in range(_ITRS):
    ht, cum = step(edg, q, ht, cum)
  return cum[:N]


# speedup vs baseline: 1.0289x; 1.0289x over previous
import functools

import jax
import jax.numpy as jnp
from jax import lax
from jax.experimental import pallas as pl
from jax.experimental.pallas import tpu as pltpu
from jax.experimental.pallas import tpu_sc as plsc

_ITRS = 7
_NC = 2
_NT = 16
_K = 128
_RSUB = 32


def _mesh():
  return plsc.VectorSubcoreMesh(
      core_axis_name="c", subcore_axis_name="s", num_cores=_NC)


def _make_phase0(N, D, E):
  NCH = E // _K
  HN = N // _NC
  RPT = HN // _NT
  NSUB = RPT // _RSUB
  CSL = D // 16

  @functools.partial(
      pl.kernel,
      out_type=(
          jax.ShapeDtypeStruct((N, D), jnp.float32),
          jax.ShapeDtypeStruct((N, D), jnp.float32),
          jax.ShapeDtypeStruct((N, D), jnp.float32),
      ),
      mesh=_mesh(),
      scratch_types=[
          pltpu.VMEM((4, _K, D), jnp.float32),
          pltpu.VMEM((4, 1, _K), jnp.int32),
          pltpu.VMEM((_RSUB, D), jnp.float32),
          pltpu.VMEM((_RSUB, D), jnp.float32),
          pltpu.VMEM((_RSUB, D), jnp.float32),
          pltpu.VMEM_SHARED((HN + 8, D), jnp.float32),
          pltpu.SemaphoreType.DMA,
          pltpu.SemaphoreType.DMA,
          pltpu.SemaphoreType.DMA,
          pltpu.SemaphoreType.DMA,
          pltpu.SemaphoreType.DMA,
          pltpu.SemaphoreType.DMA,
          pltpu.SemaphoreType.DMA,
          pltpu.SemaphoreType.DMA,
          pltpu.SemaphoreType.DMA,
          pltpu.SemaphoreType.DMA,
          pltpu.SemaphoreType.DMA,
          pltpu.SemaphoreType.DMA,
      ],
  )
  def k0(p_hbm, rel_hbm, dst3d_hbm, q_hbm, ht_hbm, cum_hbm,
         rowb, dstrow, bufA, bufQ, zbuf, agg,
         semA0, semA1, semA2, semA3, semB0, semB1, semB2, semB3,
         semC0, semC1, semC2, semC3):
    cid = lax.axis_index("c")
    w = lax.axis_index("s")
    semA = (semA0, semA1, semA2, semA3)
    semB = (semB0, semB1, semB2, semB3)
    semC = (semC0, semC1, semC2, semC3)

    def add_into(dst_ref, src_ref):
      def row(r, carry):
        for l in range(CSL):
          sl = pl.ds(l * 16, 16)
          dst_ref[r, sl] = dst_ref[r, sl] + src_ref[r, sl]
        return carry
      lax.fori_loop(0, _RSUB, row, 0)

    def zrow(r, carry):
      for l in range(CSL):
        zbuf[r, pl.ds(l * 16, 16)] = jnp.zeros((16,), jnp.float32)
      return carry
    lax.fori_loop(0, _RSUB, zrow, 0)

    def zchunk(s_, carry):
      pltpu.sync_copy(zbuf, agg.at[pl.ds(w * RPT + s_ * _RSUB, _RSUB)])
      return carry
    lax.fori_loop(0, NSUB, zchunk, 0)

    plsc.subcore_barrier()

    n_my = (NCH - w + _NT - 1) // _NT

    def rel_issue(i, u):
      j = w + i * _NT
      pltpu.async_copy(rel_hbm.at[pl.ds(j * _K, _K)], rowb.at[u], semA[u])
      pltpu.async_copy(dst3d_hbm.at[cid, j], dstrow.at[u], semB[u])

    def rel_wait(i, u):
      j = w + i * _NT
      pltpu.make_async_copy(
          rel_hbm.at[pl.ds(j * _K, _K)], rowb.at[u], semA[u]).wait()
      pltpu.make_async_copy(
          dst3d_hbm.at[cid, j], dstrow.at[u], semB[u]).wait()

    def rel_scat_issue(u):
      pltpu.async_copy(
          rowb.at[u],
          agg.at[plsc.Indices(dstrow.at[u, 0], ignored_value=-1)],
          semC[u], add=True)

    def rel_scat_wait(u):
      pltpu.make_async_copy(
          rowb.at[u],
          agg.at[plsc.Indices(dstrow.at[u, 0], ignored_value=-1)],
          semC[u]).wait()

    for u in range(4):
      @pl.when(u < n_my)
      def _():
        rel_issue(u, u)

    def relgroup(g, carry):
      for u in range(4):
        c = 4 * g + u
        @pl.when(c < n_my)
        def _():
          rel_wait(c, u)
          rel_scat_issue(u)
      for u in range(4):
        c4 = 4 * g + u + 4
        @pl.when(c4 < n_my)
        def _():
          rel_scat_wait(u)
          rel_issue(c4, u)
      return carry
    lax.fori_loop(0, (n_my + 3) // 4, relgroup, 0)

    for u in range(4):
      @pl.when(u < n_my)
      def _():
        rel_scat_wait(u)

    plsc.subcore_barrier()

    def initchunk(s_, carry):
      rows = pl.ds(w * RPT + s_ * _RSUB, _RSUB)
      grows = pl.ds(cid * HN + w * RPT + s_ * _RSUB, _RSUB)
      pltpu.sync_copy(p_hbm.at[grows], bufQ)
      pltpu.sync_copy(agg.at[rows], bufA)
      pltpu.sync_copy(bufQ, ht_hbm.at[grows])
      add_into(bufQ, bufA)
      pltpu.sync_copy(bufQ, q_hbm.at[grows])
      pltpu.sync_copy(zbuf, cum_hbm.at[grows])
      return carry
    lax.fori_loop(0, NSUB, initchunk, 0)

  return k0


def _make_iter(N, D, C):
  HN = N // _NC
  RPT = HN // _NT
  NSUB = RPT // _RSUB
  CSL = D // 16
  assert C % 4 == 0

  @functools.partial(
      pl.kernel,
      out_type=(
          jax.ShapeDtypeStruct((N, D), jnp.float32),
          jax.ShapeDtypeStruct((N, D), jnp.float32),
      ),
      mesh=_mesh(),
      scratch_types=[
          pltpu.VMEM((4, 2, _K), jnp.int32),
          pltpu.VMEM((4, _K, D), jnp.float32),
          pltpu.VMEM((_RSUB, D), jnp.float32),
          pltpu.VMEM((_RSUB, D), jnp.float32),
          pltpu.VMEM((_RSUB, D), jnp.float32),
          pltpu.VMEM_SHARED((HN + 8, D), jnp.float32),
          pltpu.SemaphoreType.DMA,
          pltpu.SemaphoreType.DMA,
          pltpu.SemaphoreType.DMA,
          pltpu.SemaphoreType.DMA,
          pltpu.SemaphoreType.DMA,
          pltpu.SemaphoreType.DMA,
          pltpu.SemaphoreType.DMA,
          pltpu.SemaphoreType.DMA,
          pltpu.SemaphoreType.DMA,
          pltpu.SemaphoreType.DMA,
          pltpu.SemaphoreType.DMA,
          pltpu.SemaphoreType.DMA,
          pltpu.SemaphoreType.DMA,
      ],
  )
  def kit(edg_hbm, q_hbm, ht_hbm, cum_hbm,
          htn_hbm, cumn_hbm,
          idxb, rowb, bufA, bufQ, zbuf, agg,
          semG0, semG1, semG2, semG3, semI0, semI1, semI2, semI3,
          semS0, semS1, semS2, semS3, semW):
    cid = lax.axis_index("c")
    w = lax.axis_index("s")
    semG = (semG0, semG1, semG2, semG3)
    semI = (semI0, semI1, semI2, semI3)
    semS = (semS0, semS1, semS2, semS3)

    def add_into(dst_ref, src_ref):
      def row(r, carry):
        for l in range(CSL):
          sl = pl.ds(l * 16, 16)
          dst_ref[r, sl] = dst_ref[r, sl] + src_ref[r, sl]
        return carry
      lax.fori_loop(0, _RSUB, row, 0)

    def zrow(r, carry):
      for l in range(CSL):
        zbuf[r, pl.ds(l * 16, 16)] = jnp.zeros((16,), jnp.float32)
      return carry
    lax.fori_loop(0, _RSUB, zrow, 0)

    def zchunk(s_, carry):
      pltpu.sync_copy(zbuf, agg.at[pl.ds(w * RPT + s_ * _RSUB, _RSUB)])
      return carry
    lax.fori_loop(0, NSUB, zchunk, 0)

    plsc.subcore_barrier()

    def idx_issue(c, u):
      pltpu.async_copy(edg_hbm.at[cid, w, c], idxb.at[u], semI[u])

    def idx_wait(c, u):
      pltpu.make_async_copy(edg_hbm.at[cid, w, c], idxb.at[u], semI[u]).wait()

    def g_issue(u):
      pltpu.async_copy(
          ht_hbm.at[plsc.Indices(idxb.at[u, 0], ignored_value=-1)],
          rowb.at[u], semG[u])

    def g_wait(u):
      pltpu.make_async_copy(
          ht_hbm.at[plsc.Indices(idxb.at[u, 0], ignored_value=-1)],
          rowb.at[u], semG[u]).wait()

    def scat_issue(u):
      pltpu.async_copy(
          rowb.at[u],
          agg.at[plsc.Indices(idxb.at[u, 1], ignored_value=-1)],
          semS[u], add=True)

    def scat_wait(u):
      pltpu.make_async_copy(
          rowb.at[u],
          agg.at[plsc.Indices(idxb.at[u, 1], ignored_value=-1)],
          semS[u]).wait()

    for u in range(4):
      idx_issue(u, u)
    for u in range(4):
      idx_wait(u, u)
      g_issue(u)

    def group(g, carry):
      for u in range(4):
        g_wait(u)
        scat_issue(u)
      for u in range(4):
        c4 = 4 * g + u + 4
        @pl.when(c4 < C)
        def _():
          scat_wait(u)
          idx_issue(c4, u)
          idx_wait(c4, u)
          g_issue(u)
      return carry
    lax.fori_loop(0, C // 4, group, 0)

    for u in range(4):
      scat_wait(u)

    plsc.subcore_barrier()

    def wait_writes(sp):
      prows = pl.ds(cid * HN + w * RPT + sp * _RSUB, _RSUB)
      pltpu.make_async_copy(bufQ, htn_hbm.at[prows], semW).wait()
      pltpu.make_async_copy(bufA, cumn_hbm.at[prows], semW).wait()

    def drain(s_, carry2):
      rows = pl.ds(w * RPT + s_ * _RSUB, _RSUB)
      grows = pl.ds(cid * HN + w * RPT + s_ * _RSUB, _RSUB)
      @pl.when(s_ > 0)
      def _():
        wait_writes(s_ - 1)
      pltpu.sync_copy(agg.at[rows], bufA)
      pltpu.sync_copy(q_hbm.at[grows], bufQ)
      add_into(bufQ, bufA)
      pltpu.async_copy(bufQ, htn_hbm.at[grows], semW)
      pltpu.sync_copy(cum_hbm.at[grows], bufA)
      add_into(bufA, bufQ)
      pltpu.async_copy(bufA, cumn_hbm.at[grows], semW)
      return carry2
    lax.fori_loop(0, NSUB, drain, 0)
    wait_writes(NSUB - 1)

  return kit


def kernel(particle_effect, relation_effect, edges):
  N, D = particle_effect.shape
  E = relation_effect.shape[0]
  assert E % _K == 0 and D % 16 == 0

  src = edges[0].astype(jnp.int32)
  dst = edges[1].astype(jnp.int32)

  NP = -(-N // (_NC * _NT * _RSUB)) * (_NC * _NT * _RSUB)
  HN = NP // _NC
  p_pad = jnp.pad(particle_effect, ((0, NP - N), (0, 0))) if NP != N \
      else particle_effect

  C = -(-E // (_NT * _K))
  C = -(-C // 8) * 8
  tot = _NT * C * _K
  pad = tot - E
  if pad:
    neg = -jnp.ones((pad,), jnp.int32)
    src_p = jnp.concatenate([src, neg])
    dst_p = jnp.concatenate([dst, neg])
  else:
    src_p, dst_p = src, dst

  own0 = (dst_p >= 0) & (dst_p < HN)
  own1 = dst_p >= HN
  def tbl(ownm, vs, vd):
    return jnp.stack([
        jnp.where(ownm, vs, -1).reshape(_NT, C, _K),
        jnp.where(ownm, vd, -1).reshape(_NT, C, _K),
    ], axis=2)
  edg = jnp.stack([tbl(own0, src_p, dst_p), tbl(own1, src_p, dst_p - HN)])

  o0 = dst < HN
  dst3d = jnp.stack([
      jnp.where(o0, dst, -1).reshape(E // _K, 1, _K),
      jnp.where(o0, -1, dst - HN).reshape(E // _K, 1, _K),
  ])

  q, ht, cum = _make_phase0(NP, D, E)(p_pad, relation_effect, dst3d)
  step = _make_iter(NP, D, C)
  for _ in range(_ITRS):
    ht, cum = step(edg, q, ht, cum)
  return cum[:N]
